# Initial kernel scaffold; baseline (speedup 1.0000x reference)
import jax
import jax.numpy as jnp
from jax.experimental import pallas as pl


def kernel(rpn_cls_prob, rpn_bbox_pred, im_info):
    raise NotImplementedError("write your pallas kernel here")


# speedup vs baseline: 138.7612x; 138.7612x over previous
import jax
import jax.numpy as jnp
import numpy as np
from jax.experimental import pallas as pl
from jax.experimental.pallas import tpu as pltpu

_FEAT_STRIDE = 16
_PRE_NMS = 6000
_POST_NMS = 300
_NMS_THRESH = 0.7

_N_REAL = 22500
_N_SORT = 32768
_ROWS = _N_SORT // 128
_S = 6144
_SROWS = _S // 128
_OUT_ROWS = 384


def _gen_anchors():
    base_size = 16
    ratios = np.array([0.5, 1.0, 2.0])
    scales = np.array([8, 16, 32])
    base = np.array([0, 0, base_size - 1, base_size - 1], dtype=np.float64)

    def whctrs(a):
        w = a[2] - a[0] + 1
        h = a[3] - a[1] + 1
        return w, h, a[0] + 0.5 * (w - 1), a[1] + 0.5 * (h - 1)

    def mk(ws, hs, xc, yc):
        ws = ws[:, None]
        hs = hs[:, None]
        return np.hstack((xc - 0.5 * (ws - 1), yc - 0.5 * (hs - 1),
                          xc + 0.5 * (ws - 1), yc + 0.5 * (hs - 1)))

    w, h, xc, yc = whctrs(base)
    size = w * h
    ws = np.round(np.sqrt(size / ratios))
    hs = np.round(ws * ratios)
    ratio_anchors = mk(ws, hs, xc, yc)
    outs = []
    for i in range(ratio_anchors.shape[0]):
        w, h, xc, yc = whctrs(ratio_anchors[i, :])
        outs.append(mk(w * scales, h * scales, xc, yc))
    return np.vstack(outs)


def _full_anchors(feat_h, feat_w):
    base = _gen_anchors()
    sx = (np.arange(feat_w) * _FEAT_STRIDE).astype(np.float64)
    sy = (np.arange(feat_h) * _FEAT_STRIDE).astype(np.float64)
    cx, cy = np.meshgrid(sx, sy)
    shifts = np.stack([cx.ravel(), cy.ravel(), cx.ravel(), cy.ravel()], axis=1)
    anchors = (base[None, :, :] + shifts[:, None, :]).reshape(-1, 4)
    return anchors.astype(np.float32)


def _sort_kernel(scores_ref, deltas_ref, anchors_ref, imhw_ref, out_ref):
    s = scores_ref[0]
    i = pl.program_id(0)
    h_im = imhw_ref[i, 0]
    w_im = imhw_ref[i, 1]

    ax1 = anchors_ref[0]
    ay1 = anchors_ref[1]
    ax2 = anchors_ref[2]
    ay2 = anchors_ref[3]
    dx = deltas_ref[0, 0]
    dy = deltas_ref[0, 1]
    dw = deltas_ref[0, 2]
    dh = deltas_ref[0, 3]

    widths = ax2 - ax1 + 1.0
    heights = ay2 - ay1 + 1.0
    ctr_x = ax1 + 0.5 * widths
    ctr_y = ay1 + 0.5 * heights
    pcx = dx * widths + ctr_x
    pcy = dy * heights + ctr_y
    pw = jnp.exp(dw) * widths
    ph = jnp.exp(dh) * heights
    x1 = jnp.clip(pcx - 0.5 * pw, 0.0, w_im - 1.0)
    y1 = jnp.clip(pcy - 0.5 * ph, 0.0, h_im - 1.0)
    x2 = jnp.clip(pcx + 0.5 * pw, 0.0, w_im - 1.0)
    y2 = jnp.clip(pcy + 0.5 * ph, 0.0, h_im - 1.0)

    row_i = jax.lax.broadcasted_iota(jnp.int32, (_ROWS, 128), 0)
    col_i = jax.lax.broadcasted_iota(jnp.int32, (_ROWS, 128), 1)
    idx = row_i * 128 + col_i

    arrays = [s, idx, x1, y1, x2, y2]

    def partner(x, j):
        if j < 128:
            sel = (col_i & j) == 0
            return jnp.where(sel, pltpu.roll(x, 128 - j, 1),
                             pltpu.roll(x, j, 1))
        dj = j // 128
        sel = (row_i & dj) == 0
        return jnp.where(sel, pltpu.roll(x, _ROWS - dj, 0),
                         pltpu.roll(x, dj, 0))

    k = 2
    while k <= _N_SORT:
        j = k // 2
        while j >= 1:
            ps = [partner(a, j) for a in arrays]
            if j < 128:
                is_lower = (col_i & j) == 0
            else:
                is_lower = (row_i & (j // 128)) == 0
            if k < 128:
                dir_up = (col_i & k) == 0
            elif k < _N_SORT:
                dir_up = (row_i & (k // 128)) == 0
            else:
                dir_up = jnp.full((_ROWS, 128), True)
            lt = (arrays[0] > ps[0]) | ((arrays[0] == ps[0]) &
                                        (arrays[1] < ps[1]))
            take_small = is_lower == dir_up
            keep_self = lt == take_small
            arrays = [jnp.where(keep_self, a, p) for a, p in zip(arrays, ps)]
            j //= 2
        k *= 2

    out_ref[0, 0] = arrays[2][:_SROWS]
    out_ref[0, 1] = arrays[3][:_SROWS]
    out_ref[0, 2] = arrays[4][:_SROWS]
    out_ref[0, 3] = arrays[5][:_SROWS]


def _nms_kernel(boxes_ref, out_ref, keep_ref, cnt_ref):
    x1 = boxes_ref[0, 0]
    y1 = boxes_ref[0, 1]
    x2 = boxes_ref[0, 2]
    y2 = boxes_ref[0, 3]
    areas = (x2 - x1 + 1.0) * (y2 - y1 + 1.0)

    keep_ref[...] = jnp.zeros((_SROWS, 128), jnp.float32)
    cnt_ref[0] = 0.0

    sub_i = jax.lax.broadcasted_iota(jnp.int32, (128, 128), 0)
    lan_i = jax.lax.broadcasted_iota(jnp.int32, (128, 128), 1)
    eye = (sub_i == lan_i).astype(jnp.float32)
    strict_lower = (lan_i < sub_i).astype(jnp.float32)

    x1b = x1[None]
    y1b = y1[None]
    x2b = x2[None]
    y2b = y2[None]
    areab = areas[None]

    def to_col(row):
        return jax.lax.dot_general(eye, row, (((1,), (1,)), ((), ())),
                                   preferred_element_type=jnp.float32)

    def to_row(col):
        return jax.lax.dot_general(col, eye, (((0,), (0,)), ((), ())),
                                   preferred_element_type=jnp.float32)

    def matvec(m, v):
        return jax.lax.dot_general(m, v, (((1,), (0,)), ((), ())),
                                   preferred_element_type=jnp.float32)

    def tile_body(t, carry):
        @pl.when(cnt_ref[0] < float(_POST_NMS))
        def _process():
            rx1 = boxes_ref[0, 0, pl.ds(t, 1), :]
            ry1 = boxes_ref[0, 1, pl.ds(t, 1), :]
            rx2 = boxes_ref[0, 2, pl.ds(t, 1), :]
            ry2 = boxes_ref[0, 3, pl.ds(t, 1), :]
            rar = (rx2 - rx1 + 1.0) * (ry2 - ry1 + 1.0)
            cx1 = to_col(rx1)
            cy1 = to_col(ry1)
            cx2 = to_col(rx2)
            cy2 = to_col(ry2)
            car = to_col(rar)

            xx1 = jnp.maximum(cx1[:, :, None], x1b)
            yy1 = jnp.maximum(cy1[:, :, None], y1b)
            xx2 = jnp.minimum(cx2[:, :, None], x2b)
            yy2 = jnp.minimum(cy2[:, :, None], y2b)
            w = jnp.maximum(0.0, xx2 - xx1 + 1.0)
            h = jnp.maximum(0.0, yy2 - yy1 + 1.0)
            inter = w * h
            iou = inter / (car[:, :, None] + areab - inter)
            m3 = (iou > _NMS_THRESH).astype(jnp.float32)

            supp = jnp.max(m3 * keep_ref[...][None], axis=(1, 2),
                           keepdims=True)[:, :, 0]
            g = t * 128 + jax.lax.broadcasted_iota(jnp.int32, (128, 1), 0)
            alive = ((supp == 0.0) & (g < _PRE_NMS)).astype(jnp.float32)

            ixx1 = jnp.maximum(cx1, rx1)
            iyy1 = jnp.maximum(cy1, ry1)
            ixx2 = jnp.minimum(cx2, rx2)
            iyy2 = jnp.minimum(cy2, ry2)
            iw = jnp.maximum(0.0, ixx2 - ixx1 + 1.0)
            ih = jnp.maximum(0.0, iyy2 - iyy1 + 1.0)
            iin = iw * ih
            iself = iin / (car + rar - iin)
            ov = jnp.where(iself > _NMS_THRESH, strict_lower, 0.0)

            lo0 = alive * jnp.where(matvec(ov, alive) > 0.0, 0.0, 1.0)
            up0 = alive

            def fx_cond(lu):
                lo, up = lu
                return jnp.sum(up - lo) > 0.0

            def fx_body(lu):
                lo, _ = lu
                up = alive * jnp.where(matvec(ov, lo) > 0.0, 0.0, 1.0)
                lo2 = alive * jnp.where(matvec(ov, up) > 0.0, 0.0, 1.0)
                return lo2, up

            lo, _ = jax.lax.while_loop(fx_cond, fx_body, (lo0, up0))

            keep_ref[pl.ds(t, 1), :] = to_row(lo)
            cnt_ref[0] = cnt_ref[0] + jnp.sum(lo)

        return carry

    jax.lax.fori_loop(0, _SROWS, tile_body, 0, unroll=False)

    keep = keep_ref[...]
    up128 = (sub_i < lan_i).astype(jnp.float32)
    inrow = jax.lax.dot_general(keep, up128, (((1,), (0,)), ((), ())),
                                preferred_element_type=jnp.float32)
    rsub = jax.lax.broadcasted_iota(jnp.int32, (_SROWS, _SROWS), 0)
    rlan = jax.lax.broadcasted_iota(jnp.int32, (_SROWS, _SROWS), 1)
    low48 = (rlan < rsub).astype(jnp.float32)
    ones_col = jnp.ones((128, 1), jnp.float32)
    rowsum = jax.lax.dot_general(keep, ones_col, (((1,), (0,)), ((), ())),
                                 preferred_element_type=jnp.float32)
    rowoff = jax.lax.dot_general(low48, rowsum, (((1,), (0,)), ((), ())),
                                 preferred_element_type=jnp.float32)
    rank = inrow + rowoff

    p_col = jax.lax.broadcasted_iota(jnp.int32, (_OUT_ROWS, 1, 1), 0)
    onehot = jnp.where((rank[None] == p_col.astype(jnp.float32)) &
                       (keep[None] > 0.0), 1.0, 0.0)

    def reduce_coord(c3):
        return jnp.sum(onehot * c3, axis=(1, 2), keepdims=True)[:, :, 0]

    ox1 = reduce_coord(x1b)
    oy1 = reduce_coord(y1b)
    ox2 = reduce_coord(x2b)
    oy2 = reduce_coord(y2b)
    bcol = jnp.full((_OUT_ROWS, 1),
                    pl.program_id(0).astype(jnp.float32))

    lane = jax.lax.broadcasted_iota(jnp.int32, (_OUT_ROWS, 128), 1)
    out = jnp.zeros((_OUT_ROWS, 128), jnp.float32)
    for c, col in enumerate([bcol, ox1, oy1, ox2, oy2]):
        out = jnp.where(lane == c, jnp.broadcast_to(col, (_OUT_ROWS, 128)),
                        out)
    out_ref[0] = out


def kernel(rpn_cls_prob, rpn_bbox_pred, im_info):
    b, _, fh, fw = rpn_cls_prob.shape
    n_real = fh * fw * 9

    scores = jnp.transpose(rpn_cls_prob[:, 9:, :, :], (0, 2, 3, 1))
    scores = scores.reshape(b, n_real)
    scores = jnp.pad(scores, ((0, 0), (0, _N_SORT - n_real)),
                     constant_values=-1.0)
    scores = scores.reshape(b, _ROWS, 128)

    deltas = jnp.transpose(rpn_bbox_pred, (0, 2, 3, 1)).reshape(b, n_real, 4)
    deltas = jnp.pad(deltas, ((0, 0), (0, _N_SORT - n_real), (0, 0)))
    deltas = jnp.transpose(deltas, (0, 2, 1)).reshape(b, 4, _ROWS, 128)

    anchors = _full_anchors(fh, fw)
    anchors = np.pad(anchors, ((0, _N_SORT - n_real), (0, 0)))
    anchors = jnp.asarray(anchors.T.reshape(4, _ROWS, 128))

    imhw = im_info[:, :2]

    boxes = pl.pallas_call(
        _sort_kernel,
        grid=(b,),
        in_specs=[
            pl.BlockSpec((1, _ROWS, 128), lambda i: (i, 0, 0)),
            pl.BlockSpec((1, 4, _ROWS, 128), lambda i: (i, 0, 0, 0)),
            pl.BlockSpec((4, _ROWS, 128), lambda i: (0, 0, 0)),
            pl.BlockSpec((4, 2), lambda i: (0, 0),
                         memory_space=pltpu.SMEM),
        ],
        out_specs=pl.BlockSpec((1, 4, _SROWS, 128), lambda i: (i, 0, 0, 0)),
        out_shape=jax.ShapeDtypeStruct((b, 4, _SROWS, 128), jnp.float32),
    )(scores, deltas, anchors, imhw)

    out = pl.pallas_call(
        _nms_kernel,
        grid=(b,),
        in_specs=[
            pl.BlockSpec((1, 4, _SROWS, 128), lambda i: (i, 0, 0, 0)),
        ],
        out_specs=pl.BlockSpec((1, _OUT_ROWS, 128), lambda i: (i, 0, 0)),
        out_shape=jax.ShapeDtypeStruct((b, _OUT_ROWS, 128), jnp.float32),
        scratch_shapes=[
            pltpu.VMEM((_SROWS, 128), jnp.float32),
            pltpu.SMEM((1,), jnp.float32),
        ],
    )(boxes)

    return out[:, :_POST_NMS, :5]


# speedup vs baseline: 174.0184x; 1.2541x over previous
import functools

import jax
import jax.numpy as jnp
import numpy as np
from jax import lax
from jax.experimental import pallas as pl
from jax.experimental.pallas import tpu as pltpu
from jax.experimental.pallas import tpu_sc as plsc

_FEAT_STRIDE = 16
_PRE_NMS = 6000
_POST_NMS = 300
_NMS_THRESH = 0.7

_N_REAL = 22500
_N_SORT = 32768
_ROWS = _N_SORT // 128
_S = 6144
_SROWS = _S // 128
_OUT_ROWS = 384
_N_TAB = 22528
_TROWS = _N_TAB // 128


def _gen_anchors():
    base_size = 16
    ratios = np.array([0.5, 1.0, 2.0])
    scales = np.array([8, 16, 32])
    base = np.array([0, 0, base_size - 1, base_size - 1], dtype=np.float64)

    def whctrs(a):
        w = a[2] - a[0] + 1
        h = a[3] - a[1] + 1
        return w, h, a[0] + 0.5 * (w - 1), a[1] + 0.5 * (h - 1)

    def mk(ws, hs, xc, yc):
        ws = ws[:, None]
        hs = hs[:, None]
        return np.hstack((xc - 0.5 * (ws - 1), yc - 0.5 * (hs - 1),
                          xc + 0.5 * (ws - 1), yc + 0.5 * (hs - 1)))

    w, h, xc, yc = whctrs(base)
    size = w * h
    ws = np.round(np.sqrt(size / ratios))
    hs = np.round(ws * ratios)
    ratio_anchors = mk(ws, hs, xc, yc)
    outs = []
    for i in range(ratio_anchors.shape[0]):
        w, h, xc, yc = whctrs(ratio_anchors[i, :])
        outs.append(mk(w * scales, h * scales, xc, yc))
    return np.vstack(outs)


def _full_anchors(feat_h, feat_w):
    base = _gen_anchors()
    sx = (np.arange(feat_w) * _FEAT_STRIDE).astype(np.float64)
    sy = (np.arange(feat_h) * _FEAT_STRIDE).astype(np.float64)
    cx, cy = np.meshgrid(sx, sy)
    shifts = np.stack([cx.ravel(), cy.ravel(), cx.ravel(), cy.ravel()], axis=1)
    anchors = (base[None, :, :] + shifts[:, None, :]).reshape(-1, 4)
    return anchors.astype(np.float32)


def _sort_kernel(scores_ref, deltas_ref, anchors_ref, imhw_ref,
                 table_ref, idx_ref):
    s = scores_ref[0]
    i = pl.program_id(0)
    h_im = imhw_ref[i, 0]
    w_im = imhw_ref[i, 1]

    ax1 = anchors_ref[0]
    ay1 = anchors_ref[1]
    ax2 = anchors_ref[2]
    ay2 = anchors_ref[3]
    dx = deltas_ref[0, 0]
    dy = deltas_ref[0, 1]
    dw = deltas_ref[0, 2]
    dh = deltas_ref[0, 3]

    widths = ax2 - ax1 + 1.0
    heights = ay2 - ay1 + 1.0
    ctr_x = ax1 + 0.5 * widths
    ctr_y = ay1 + 0.5 * heights
    pcx = dx * widths + ctr_x
    pcy = dy * heights + ctr_y
    pw = jnp.exp(dw) * widths
    ph = jnp.exp(dh) * heights
    x1 = jnp.clip(pcx - 0.5 * pw, 0.0, w_im - 1.0)
    y1 = jnp.clip(pcy - 0.5 * ph, 0.0, h_im - 1.0)
    x2 = jnp.clip(pcx + 0.5 * pw, 0.0, w_im - 1.0)
    y2 = jnp.clip(pcy + 0.5 * ph, 0.0, h_im - 1.0)

    table_ref[0, 0] = x1[:_TROWS]
    table_ref[0, 1] = y1[:_TROWS]
    table_ref[0, 2] = x2[:_TROWS]
    table_ref[0, 3] = y2[:_TROWS]

    row_i = jax.lax.broadcasted_iota(jnp.int32, (_ROWS, 128), 0)
    col_i = jax.lax.broadcasted_iota(jnp.int32, (_ROWS, 128), 1)
    idx = row_i * 128 + col_i

    arrays = [s, idx]

    def partner(x, j):
        if j < 128:
            sel = (col_i & j) == 0
            return jnp.where(sel, pltpu.roll(x, 128 - j, 1),
                             pltpu.roll(x, j, 1))
        dj = j // 128
        sel = (row_i & dj) == 0
        return jnp.where(sel, pltpu.roll(x, _ROWS - dj, 0),
                         pltpu.roll(x, dj, 0))

    k = 2
    while k <= _N_SORT:
        j = k // 2
        while j >= 1:
            ps = [partner(a, j) for a in arrays]
            if j < 128:
                is_lower = (col_i & j) == 0
            else:
                is_lower = (row_i & (j // 128)) == 0
            if k < 128:
                dir_up = (col_i & k) == 0
            elif k < _N_SORT:
                dir_up = (row_i & (k // 128)) == 0
            else:
                dir_up = jnp.full((_ROWS, 128), True)
            lt = (arrays[0] > ps[0]) | ((arrays[0] == ps[0]) &
                                        (arrays[1] < ps[1]))
            take_small = is_lower == dir_up
            keep_self = lt == take_small
            arrays = [jnp.where(keep_self, a, p) for a, p in zip(arrays, ps)]
            j //= 2
        k *= 2

    sidx = arrays[1][:_SROWS]
    base = (i * 4) * _N_TAB
    for c in range(4):
        idx_ref[0, c] = base + c * _N_TAB + sidx


def _make_sc_gather(b):
    n_idx = b * 4 * _S
    nw = 32
    per_w = n_idx // nw
    chunks = per_w // 128
    mesh = plsc.VectorSubcoreMesh(core_axis_name="c", subcore_axis_name="s")

    @functools.partial(
        pl.kernel,
        out_type=jax.ShapeDtypeStruct((n_idx,), jnp.float32),
        mesh=mesh,
        scratch_types=[
            pltpu.VMEM((128,), jnp.int32),
            pltpu.VMEM((128,), jnp.float32),
            pltpu.SemaphoreType.DMA,
        ],
    )
    def sc_gather(table_hbm, idx_hbm, out_hbm, idx_v, rows_v, sem):
        wid = lax.axis_index("s") * 2 + lax.axis_index("c")
        base = wid * per_w
        for j in range(chunks):
            off = base + j * 128
            pltpu.sync_copy(idx_hbm.at[pl.ds(off, 128)], idx_v)
            pltpu.async_copy(table_hbm.at[idx_v], rows_v, sem).wait()
            pltpu.sync_copy(rows_v, out_hbm.at[pl.ds(off, 128)])

    return sc_gather


def _nms_kernel(boxes_ref, out_ref, keep_ref, cnt_ref):
    x1 = boxes_ref[0, 0]
    y1 = boxes_ref[0, 1]
    x2 = boxes_ref[0, 2]
    y2 = boxes_ref[0, 3]
    areas = (x2 - x1 + 1.0) * (y2 - y1 + 1.0)

    keep_ref[...] = jnp.zeros((_SROWS, 128), jnp.float32)
    cnt_ref[0] = 0.0

    sub_i = jax.lax.broadcasted_iota(jnp.int32, (128, 128), 0)
    lan_i = jax.lax.broadcasted_iota(jnp.int32, (128, 128), 1)
    eye = (sub_i == lan_i).astype(jnp.float32)
    strict_lower = (lan_i < sub_i).astype(jnp.float32)

    x1b = x1[None]
    y1b = y1[None]
    x2b = x2[None]
    y2b = y2[None]
    areab = areas[None]

    def to_col(row):
        return jax.lax.dot_general(eye, row, (((1,), (1,)), ((), ())),
                                   preferred_element_type=jnp.float32)

    def to_row(col):
        return jax.lax.dot_general(col, eye, (((0,), (0,)), ((), ())),
                                   preferred_element_type=jnp.float32)

    def matvec(m, v):
        return jax.lax.dot_general(m, v, (((1,), (0,)), ((), ())),
                                   preferred_element_type=jnp.float32)

    def tile_body(t, carry):
        @pl.when(cnt_ref[0] < float(_POST_NMS))
        def _process():
            rx1 = boxes_ref[0, 0, pl.ds(t, 1), :]
            ry1 = boxes_ref[0, 1, pl.ds(t, 1), :]
            rx2 = boxes_ref[0, 2, pl.ds(t, 1), :]
            ry2 = boxes_ref[0, 3, pl.ds(t, 1), :]
            rar = (rx2 - rx1 + 1.0) * (ry2 - ry1 + 1.0)
            cx1 = to_col(rx1)
            cy1 = to_col(ry1)
            cx2 = to_col(rx2)
            cy2 = to_col(ry2)
            car = to_col(rar)

            xx1 = jnp.maximum(cx1[:, :, None], x1b)
            yy1 = jnp.maximum(cy1[:, :, None], y1b)
            xx2 = jnp.minimum(cx2[:, :, None], x2b)
            yy2 = jnp.minimum(cy2[:, :, None], y2b)
            w = jnp.maximum(0.0, xx2 - xx1 + 1.0)
            h = jnp.maximum(0.0, yy2 - yy1 + 1.0)
            inter = w * h
            iou = inter / (car[:, :, None] + areab - inter)
            m3 = (iou > _NMS_THRESH).astype(jnp.float32)

            supp = jnp.max(m3 * keep_ref[...][None], axis=(1, 2),
                           keepdims=True)[:, :, 0]
            g = t * 128 + jax.lax.broadcasted_iota(jnp.int32, (128, 1), 0)
            alive = ((supp == 0.0) & (g < _PRE_NMS)).astype(jnp.float32)

            ixx1 = jnp.maximum(cx1, rx1)
            iyy1 = jnp.maximum(cy1, ry1)
            ixx2 = jnp.minimum(cx2, rx2)
            iyy2 = jnp.minimum(cy2, ry2)
            iw = jnp.maximum(0.0, ixx2 - ixx1 + 1.0)
            ih = jnp.maximum(0.0, iyy2 - iyy1 + 1.0)
            iin = iw * ih
            iself = iin / (car + rar - iin)
            ov = jnp.where(iself > _NMS_THRESH, strict_lower, 0.0)

            lo0 = alive * jnp.where(matvec(ov, alive) > 0.0, 0.0, 1.0)
            up0 = alive

            def fx_cond(lu):
                lo, up = lu
                return jnp.sum(up - lo) > 0.0

            def fx_body(lu):
                lo, _ = lu
                up = alive * jnp.where(matvec(ov, lo) > 0.0, 0.0, 1.0)
                lo2 = alive * jnp.where(matvec(ov, up) > 0.0, 0.0, 1.0)
                return lo2, up

            lo, _ = jax.lax.while_loop(fx_cond, fx_body, (lo0, up0))

            keep_ref[pl.ds(t, 1), :] = to_row(lo)
            cnt_ref[0] = cnt_ref[0] + jnp.sum(lo)

        return carry

    jax.lax.fori_loop(0, _SROWS, tile_body, 0, unroll=False)

    keep = keep_ref[...]
    up128 = (sub_i < lan_i).astype(jnp.float32)
    inrow = jax.lax.dot_general(keep, up128, (((1,), (0,)), ((), ())),
                                preferred_element_type=jnp.float32)
    rsub = jax.lax.broadcasted_iota(jnp.int32, (_SROWS, _SROWS), 0)
    rlan = jax.lax.broadcasted_iota(jnp.int32, (_SROWS, _SROWS), 1)
    low48 = (rlan < rsub).astype(jnp.float32)
    ones_col = jnp.ones((128, 1), jnp.float32)
    rowsum = jax.lax.dot_general(keep, ones_col, (((1,), (0,)), ((), ())),
                                 preferred_element_type=jnp.float32)
    rowoff = jax.lax.dot_general(low48, rowsum, (((1,), (0,)), ((), ())),
                                 preferred_element_type=jnp.float32)
    rank = inrow + rowoff

    p_col = jax.lax.broadcasted_iota(jnp.int32, (_OUT_ROWS, 1, 1), 0)
    onehot = jnp.where((rank[None] == p_col.astype(jnp.float32)) &
                       (keep[None] > 0.0), 1.0, 0.0)

    def reduce_coord(c3):
        return jnp.sum(onehot * c3, axis=(1, 2), keepdims=True)[:, :, 0]

    ox1 = reduce_coord(x1b)
    oy1 = reduce_coord(y1b)
    ox2 = reduce_coord(x2b)
    oy2 = reduce_coord(y2b)
    bcol = jnp.full((_OUT_ROWS, 1),
                    pl.program_id(0).astype(jnp.float32))

    lane = jax.lax.broadcasted_iota(jnp.int32, (_OUT_ROWS, 128), 1)
    out = jnp.zeros((_OUT_ROWS, 128), jnp.float32)
    for c, col in enumerate([bcol, ox1, oy1, ox2, oy2]):
        out = jnp.where(lane == c, jnp.broadcast_to(col, (_OUT_ROWS, 128)),
                        out)
    out_ref[0] = out


def kernel(rpn_cls_prob, rpn_bbox_pred, im_info):
    b, _, fh, fw = rpn_cls_prob.shape
    n_real = fh * fw * 9

    scores = jnp.transpose(rpn_cls_prob[:, 9:, :, :], (0, 2, 3, 1))
    scores = scores.reshape(b, n_real)
    scores = jnp.pad(scores, ((0, 0), (0, _N_SORT - n_real)),
                     constant_values=-1.0)
    scores = scores.reshape(b, _ROWS, 128)

    deltas = jnp.transpose(rpn_bbox_pred, (0, 2, 3, 1)).reshape(b, n_real, 4)
    deltas = jnp.pad(deltas, ((0, 0), (0, _N_SORT - n_real), (0, 0)))
    deltas = jnp.transpose(deltas, (0, 2, 1)).reshape(b, 4, _ROWS, 128)

    anchors = _full_anchors(fh, fw)
    anchors = np.pad(anchors, ((0, _N_SORT - n_real), (0, 0)))
    anchors = jnp.asarray(anchors.T.reshape(4, _ROWS, 128))

    imhw = im_info[:, :2]

    table, flatidx = pl.pallas_call(
        _sort_kernel,
        grid=(b,),
        in_specs=[
            pl.BlockSpec((1, _ROWS, 128), lambda i: (i, 0, 0)),
            pl.BlockSpec((1, 4, _ROWS, 128), lambda i: (i, 0, 0, 0)),
            pl.BlockSpec((4, _ROWS, 128), lambda i: (0, 0, 0)),
            pl.BlockSpec((4, 2), lambda i: (0, 0),
                         memory_space=pltpu.SMEM),
        ],
        out_specs=(
            pl.BlockSpec((1, 4, _TROWS, 128), lambda i: (i, 0, 0, 0)),
            pl.BlockSpec((1, 4, _SROWS, 128), lambda i: (i, 0, 0, 0)),
        ),
        out_shape=(
            jax.ShapeDtypeStruct((b, 4, _TROWS, 128), jnp.float32),
            jax.ShapeDtypeStruct((b, 4, _SROWS, 128), jnp.int32),
        ),
    )(scores, deltas, anchors, imhw)

    gathered = _make_sc_gather(b)(table.reshape(-1), flatidx.reshape(-1))
    boxes = gathered.reshape(b, 4, _SROWS, 128)

    out = pl.pallas_call(
        _nms_kernel,
        grid=(b,),
        in_specs=[
            pl.BlockSpec((1, 4, _SROWS, 128), lambda i: (i, 0, 0, 0)),
        ],
        out_specs=pl.BlockSpec((1, _OUT_ROWS, 128), lambda i: (i, 0, 0)),
        out_shape=jax.ShapeDtypeStruct((b, _OUT_ROWS, 128), jnp.float32),
        scratch_shapes=[
            pltpu.VMEM((_SROWS, 128), jnp.float32),
            pltpu.SMEM((1,), jnp.float32),
        ],
    )(boxes)

    return out[:, :_POST_NMS, :5]


# speedup vs baseline: 193.1045x; 1.1097x over previous
import functools

import jax
import jax.numpy as jnp
import numpy as np
from jax import lax
from jax.experimental import pallas as pl
from jax.experimental.pallas import tpu as pltpu
from jax.experimental.pallas import tpu_sc as plsc

_FEAT_STRIDE = 16
_PRE_NMS = 6000
_POST_NMS = 300
_NMS_THRESH = 0.7

_N_REAL = 22500
_N_SORT = 32768
_ROWS = _N_SORT // 128
_S = 6144
_SROWS = _S // 128
_OUT_ROWS = 384
_N_TAB = 22528
_TROWS = _N_TAB // 128


def _gen_anchors():
    base_size = 16
    ratios = np.array([0.5, 1.0, 2.0])
    scales = np.array([8, 16, 32])
    base = np.array([0, 0, base_size - 1, base_size - 1], dtype=np.float64)

    def whctrs(a):
        w = a[2] - a[0] + 1
        h = a[3] - a[1] + 1
        return w, h, a[0] + 0.5 * (w - 1), a[1] + 0.5 * (h - 1)

    def mk(ws, hs, xc, yc):
        ws = ws[:, None]
        hs = hs[:, None]
        return np.hstack((xc - 0.5 * (ws - 1), yc - 0.5 * (hs - 1),
                          xc + 0.5 * (ws - 1), yc + 0.5 * (hs - 1)))

    w, h, xc, yc = whctrs(base)
    size = w * h
    ws = np.round(np.sqrt(size / ratios))
    hs = np.round(ws * ratios)
    ratio_anchors = mk(ws, hs, xc, yc)
    outs = []
    for i in range(ratio_anchors.shape[0]):
        w, h, xc, yc = whctrs(ratio_anchors[i, :])
        outs.append(mk(w * scales, h * scales, xc, yc))
    return np.vstack(outs)


def _full_anchors(feat_h, feat_w):
    base = _gen_anchors()
    sx = (np.arange(feat_w) * _FEAT_STRIDE).astype(np.float64)
    sy = (np.arange(feat_h) * _FEAT_STRIDE).astype(np.float64)
    cx, cy = np.meshgrid(sx, sy)
    shifts = np.stack([cx.ravel(), cy.ravel(), cx.ravel(), cy.ravel()], axis=1)
    anchors = (base[None, :, :] + shifts[:, None, :]).reshape(-1, 4)
    return anchors.astype(np.float32)


def _sort_kernel(scores_ref, deltas_ref, anchors_ref, imhw_ref,
                 table_ref, idx_ref):
    s = scores_ref[0]
    i = pl.program_id(0)
    h_im = imhw_ref[i, 0]
    w_im = imhw_ref[i, 1]

    ax1 = anchors_ref[0]
    ay1 = anchors_ref[1]
    ax2 = anchors_ref[2]
    ay2 = anchors_ref[3]
    dx = deltas_ref[0, 0]
    dy = deltas_ref[0, 1]
    dw = deltas_ref[0, 2]
    dh = deltas_ref[0, 3]

    widths = ax2 - ax1 + 1.0
    heights = ay2 - ay1 + 1.0
    ctr_x = ax1 + 0.5 * widths
    ctr_y = ay1 + 0.5 * heights
    pcx = dx * widths + ctr_x
    pcy = dy * heights + ctr_y
    pw = jnp.exp(dw) * widths
    ph = jnp.exp(dh) * heights
    x1 = jnp.clip(pcx - 0.5 * pw, 0.0, w_im - 1.0)
    y1 = jnp.clip(pcy - 0.5 * ph, 0.0, h_im - 1.0)
    x2 = jnp.clip(pcx + 0.5 * pw, 0.0, w_im - 1.0)
    y2 = jnp.clip(pcy + 0.5 * ph, 0.0, h_im - 1.0)

    table_ref[0, 0] = x1[:_TROWS]
    table_ref[0, 1] = y1[:_TROWS]
    table_ref[0, 2] = x2[:_TROWS]
    table_ref[0, 3] = y2[:_TROWS]

    row_i = jax.lax.broadcasted_iota(jnp.int32, (_ROWS, 128), 0)
    col_i = jax.lax.broadcasted_iota(jnp.int32, (_ROWS, 128), 1)
    idx = row_i * 128 + col_i

    arrays = [s, idx]

    def partner(x, j):
        if j < 128:
            sel = (col_i & j) == 0
            return jnp.where(sel, pltpu.roll(x, 128 - j, 1),
                             pltpu.roll(x, j, 1))
        dj = j // 128
        sel = (row_i & dj) == 0
        return jnp.where(sel, pltpu.roll(x, _ROWS - dj, 0),
                         pltpu.roll(x, dj, 0))

    k = 2
    while k <= _N_SORT:
        j = k // 2
        while j >= 1:
            ps = [partner(a, j) for a in arrays]
            if j < 128:
                is_lower = (col_i & j) == 0
            else:
                is_lower = (row_i & (j // 128)) == 0
            if k < 128:
                dir_up = (col_i & k) == 0
            elif k < _N_SORT:
                dir_up = (row_i & (k // 128)) == 0
            else:
                dir_up = jnp.full((_ROWS, 128), True)
            lt = (arrays[0] > ps[0]) | ((arrays[0] == ps[0]) &
                                        (arrays[1] < ps[1]))
            take_small = is_lower == dir_up
            keep_self = lt == take_small
            arrays = [jnp.where(keep_self, a, p) for a, p in zip(arrays, ps)]
            j //= 2
        k *= 2

    sidx = arrays[1][:_SROWS]
    base = (i * 4) * _N_TAB
    for c in range(4):
        idx_ref[0, c] = base + c * _N_TAB + sidx


def _make_sc_gather(b):
    n_idx = b * 4 * _S
    nw = 32
    per_w = n_idx // nw
    chunks = per_w // 128
    mesh = plsc.VectorSubcoreMesh(core_axis_name="c", subcore_axis_name="s")

    @functools.partial(
        pl.kernel,
        out_type=jax.ShapeDtypeStruct((n_idx,), jnp.float32),
        mesh=mesh,
        scratch_types=[
            pltpu.VMEM((per_w,), jnp.int32),
            pltpu.VMEM((per_w,), jnp.float32),
            pltpu.SemaphoreType.DMA,
        ],
    )
    def sc_gather(table_hbm, idx_hbm, out_hbm, idx_v, rows_v, sem):
        wid = lax.axis_index("s") * 2 + lax.axis_index("c")
        base = wid * per_w
        pltpu.sync_copy(idx_hbm.at[pl.ds(base, per_w)], idx_v)
        copies = [
            pltpu.async_copy(table_hbm.at[idx_v.at[pl.ds(j * 128, 128)]],
                             rows_v.at[pl.ds(j * 128, 128)], sem)
            for j in range(chunks)
        ]
        for cp in copies:
            cp.wait()
        pltpu.sync_copy(rows_v, out_hbm.at[pl.ds(base, per_w)])

    return sc_gather


def _nms_kernel(boxes_ref, out_ref, keep_ref, cnt_ref):
    x1 = boxes_ref[0, 0]
    y1 = boxes_ref[0, 1]
    x2 = boxes_ref[0, 2]
    y2 = boxes_ref[0, 3]
    areas = (x2 - x1 + 1.0) * (y2 - y1 + 1.0)

    keep_ref[...] = jnp.zeros((_SROWS, 128), jnp.float32)
    cnt_ref[0] = 0.0

    sub_i = jax.lax.broadcasted_iota(jnp.int32, (128, 128), 0)
    lan_i = jax.lax.broadcasted_iota(jnp.int32, (128, 128), 1)
    eye = (sub_i == lan_i).astype(jnp.float32)
    strict_lower = (lan_i < sub_i).astype(jnp.float32)

    x1b = x1[None]
    y1b = y1[None]
    x2b = x2[None]
    y2b = y2[None]
    areab = areas[None]

    def to_col(row):
        return jax.lax.dot_general(eye, row, (((1,), (1,)), ((), ())),
                                   preferred_element_type=jnp.float32)

    def to_row(col):
        return jax.lax.dot_general(col, eye, (((0,), (0,)), ((), ())),
                                   preferred_element_type=jnp.float32)

    def matvec(m, v):
        return jax.lax.dot_general(m, v, (((1,), (0,)), ((), ())),
                                   preferred_element_type=jnp.float32)

    def tile_body(t, carry):
        @pl.when(cnt_ref[0] < float(_POST_NMS))
        def _process():
            rx1 = boxes_ref[0, 0, pl.ds(t, 1), :]
            ry1 = boxes_ref[0, 1, pl.ds(t, 1), :]
            rx2 = boxes_ref[0, 2, pl.ds(t, 1), :]
            ry2 = boxes_ref[0, 3, pl.ds(t, 1), :]
            rar = (rx2 - rx1 + 1.0) * (ry2 - ry1 + 1.0)
            cx1 = to_col(rx1)
            cy1 = to_col(ry1)
            cx2 = to_col(rx2)
            cy2 = to_col(ry2)
            car = to_col(rar)

            xx1 = jnp.maximum(cx1[:, :, None], x1b)
            yy1 = jnp.maximum(cy1[:, :, None], y1b)
            xx2 = jnp.minimum(cx2[:, :, None], x2b)
            yy2 = jnp.minimum(cy2[:, :, None], y2b)
            w = jnp.maximum(0.0, xx2 - xx1 + 1.0)
            h = jnp.maximum(0.0, yy2 - yy1 + 1.0)
            inter = w * h
            iou = inter / (car[:, :, None] + areab - inter)
            m3 = (iou > _NMS_THRESH).astype(jnp.float32)

            supp = jnp.max(m3 * keep_ref[...][None], axis=(1, 2),
                           keepdims=True)[:, :, 0]
            g = t * 128 + jax.lax.broadcasted_iota(jnp.int32, (128, 1), 0)
            alive = ((supp == 0.0) & (g < _PRE_NMS)).astype(jnp.float32)

            ixx1 = jnp.maximum(cx1, rx1)
            iyy1 = jnp.maximum(cy1, ry1)
            ixx2 = jnp.minimum(cx2, rx2)
            iyy2 = jnp.minimum(cy2, ry2)
            iw = jnp.maximum(0.0, ixx2 - ixx1 + 1.0)
            ih = jnp.maximum(0.0, iyy2 - iyy1 + 1.0)
            iin = iw * ih
            iself = iin / (car + rar - iin)
            ov = jnp.where(iself > _NMS_THRESH, strict_lower, 0.0)

            lo0 = alive * jnp.where(matvec(ov, alive) > 0.0, 0.0, 1.0)
            up0 = alive

            def fx_cond(lu):
                lo, up = lu
                return jnp.sum(up - lo) > 0.0

            def fx_body(lu):
                lo, _ = lu
                up = alive * jnp.where(matvec(ov, lo) > 0.0, 0.0, 1.0)
                lo2 = alive * jnp.where(matvec(ov, up) > 0.0, 0.0, 1.0)
                return lo2, up

            lo, _ = jax.lax.while_loop(fx_cond, fx_body, (lo0, up0))

            keep_ref[pl.ds(t, 1), :] = to_row(lo)
            cnt_ref[0] = cnt_ref[0] + jnp.sum(lo)

        return carry

    jax.lax.fori_loop(0, _SROWS, tile_body, 0, unroll=False)

    keep = keep_ref[...]
    up128 = (sub_i < lan_i).astype(jnp.float32)
    inrow = jax.lax.dot_general(keep, up128, (((1,), (0,)), ((), ())),
                                preferred_element_type=jnp.float32)
    rsub = jax.lax.broadcasted_iota(jnp.int32, (_SROWS, _SROWS), 0)
    rlan = jax.lax.broadcasted_iota(jnp.int32, (_SROWS, _SROWS), 1)
    low48 = (rlan < rsub).astype(jnp.float32)
    ones_col = jnp.ones((128, 1), jnp.float32)
    rowsum = jax.lax.dot_general(keep, ones_col, (((1,), (0,)), ((), ())),
                                 preferred_element_type=jnp.float32)
    rowoff = jax.lax.dot_general(low48, rowsum, (((1,), (0,)), ((), ())),
                                 preferred_element_type=jnp.float32)
    rank = inrow + rowoff

    p_col = jax.lax.broadcasted_iota(jnp.int32, (_OUT_ROWS, 1, 1), 0)
    onehot = jnp.where((rank[None] == p_col.astype(jnp.float32)) &
                       (keep[None] > 0.0), 1.0, 0.0)

    def reduce_coord(c3):
        return jnp.sum(onehot * c3, axis=(1, 2), keepdims=True)[:, :, 0]

    ox1 = reduce_coord(x1b)
    oy1 = reduce_coord(y1b)
    ox2 = reduce_coord(x2b)
    oy2 = reduce_coord(y2b)
    bcol = jnp.full((_OUT_ROWS, 1),
                    pl.program_id(0).astype(jnp.float32))

    lane = jax.lax.broadcasted_iota(jnp.int32, (_OUT_ROWS, 128), 1)
    out = jnp.zeros((_OUT_ROWS, 128), jnp.float32)
    for c, col in enumerate([bcol, ox1, oy1, ox2, oy2]):
        out = jnp.where(lane == c, jnp.broadcast_to(col, (_OUT_ROWS, 128)),
                        out)
    out_ref[0] = out


def kernel(rpn_cls_prob, rpn_bbox_pred, im_info):
    b, _, fh, fw = rpn_cls_prob.shape
    n_real = fh * fw * 9

    scores = jnp.transpose(rpn_cls_prob[:, 9:, :, :], (0, 2, 3, 1))
    scores = scores.reshape(b, n_real)
    scores = jnp.pad(scores, ((0, 0), (0, _N_SORT - n_real)),
                     constant_values=-1.0)
    scores = scores.reshape(b, _ROWS, 128)

    deltas = jnp.transpose(rpn_bbox_pred, (0, 2, 3, 1)).reshape(b, n_real, 4)
    deltas = jnp.pad(deltas, ((0, 0), (0, _N_SORT - n_real), (0, 0)))
    deltas = jnp.transpose(deltas, (0, 2, 1)).reshape(b, 4, _ROWS, 128)

    anchors = _full_anchors(fh, fw)
    anchors = np.pad(anchors, ((0, _N_SORT - n_real), (0, 0)))
    anchors = jnp.asarray(anchors.T.reshape(4, _ROWS, 128))

    imhw = im_info[:, :2]

    table, flatidx = pl.pallas_call(
        _sort_kernel,
        grid=(b,),
        in_specs=[
            pl.BlockSpec((1, _ROWS, 128), lambda i: (i, 0, 0)),
            pl.BlockSpec((1, 4, _ROWS, 128), lambda i: (i, 0, 0, 0)),
            pl.BlockSpec((4, _ROWS, 128), lambda i: (0, 0, 0)),
            pl.BlockSpec((4, 2), lambda i: (0, 0),
                         memory_space=pltpu.SMEM),
        ],
        out_specs=(
            pl.BlockSpec((1, 4, _TROWS, 128), lambda i: (i, 0, 0, 0)),
            pl.BlockSpec((1, 4, _SROWS, 128), lambda i: (i, 0, 0, 0)),
        ),
        out_shape=(
            jax.ShapeDtypeStruct((b, 4, _TROWS, 128), jnp.float32),
            jax.ShapeDtypeStruct((b, 4, _SROWS, 128), jnp.int32),
        ),
    )(scores, deltas, anchors, imhw)

    gathered = _make_sc_gather(b)(table.reshape(-1), flatidx.reshape(-1))
    boxes = gathered.reshape(b, 4, _SROWS, 128)

    out = pl.pallas_call(
        _nms_kernel,
        grid=(b,),
        in_specs=[
            pl.BlockSpec((1, 4, _SROWS, 128), lambda i: (i, 0, 0, 0)),
        ],
        out_specs=pl.BlockSpec((1, _OUT_ROWS, 128), lambda i: (i, 0, 0)),
        out_shape=jax.ShapeDtypeStruct((b, _OUT_ROWS, 128), jnp.float32),
        scratch_shapes=[
            pltpu.VMEM((_SROWS, 128), jnp.float32),
            pltpu.SMEM((1,), jnp.float32),
        ],
    )(boxes)

    return out[:, :_POST_NMS, :5]


# speedup vs baseline: 309.4970x; 1.6027x over previous
import functools

import jax
import jax.numpy as jnp
import numpy as np
from jax import lax
from jax.experimental import pallas as pl
from jax.experimental.pallas import tpu as pltpu
from jax.experimental.pallas import tpu_sc as plsc

_FEAT_STRIDE = 16
_PRE_NMS = 6000
_POST_NMS = 300
_NMS_THRESH = 0.7

_N_REAL = 22500
_N_SORT = 32768
_ROWS = _N_SORT // 128
_S = 6144
_SROWS = _S // 128
_OUT_ROWS = 384
_N_TAB = 22528
_TROWS = _N_TAB // 128


def _gen_anchors():
    base_size = 16
    ratios = np.array([0.5, 1.0, 2.0])
    scales = np.array([8, 16, 32])
    base = np.array([0, 0, base_size - 1, base_size - 1], dtype=np.float64)

    def whctrs(a):
        w = a[2] - a[0] + 1
        h = a[3] - a[1] + 1
        return w, h, a[0] + 0.5 * (w - 1), a[1] + 0.5 * (h - 1)

    def mk(ws, hs, xc, yc):
        ws = ws[:, None]
        hs = hs[:, None]
        return np.hstack((xc - 0.5 * (ws - 1), yc - 0.5 * (hs - 1),
                          xc + 0.5 * (ws - 1), yc + 0.5 * (hs - 1)))

    w, h, xc, yc = whctrs(base)
    size = w * h
    ws = np.round(np.sqrt(size / ratios))
    hs = np.round(ws * ratios)
    ratio_anchors = mk(ws, hs, xc, yc)
    outs = []
    for i in range(ratio_anchors.shape[0]):
        w, h, xc, yc = whctrs(ratio_anchors[i, :])
        outs.append(mk(w * scales, h * scales, xc, yc))
    return np.vstack(outs)


def _full_anchors(feat_h, feat_w):
    base = _gen_anchors()
    sx = (np.arange(feat_w) * _FEAT_STRIDE).astype(np.float64)
    sy = (np.arange(feat_h) * _FEAT_STRIDE).astype(np.float64)
    cx, cy = np.meshgrid(sx, sy)
    shifts = np.stack([cx.ravel(), cy.ravel(), cx.ravel(), cy.ravel()], axis=1)
    anchors = (base[None, :, :] + shifts[:, None, :]).reshape(-1, 4)
    return anchors.astype(np.float32)


def _sort_kernel(scores_ref, deltas_ref, anchors_ref, imhw_ref,
                 table_ref, idx_ref):
    s = scores_ref[0]
    i = pl.program_id(0)
    h_im = imhw_ref[i, 0]
    w_im = imhw_ref[i, 1]

    ax1 = anchors_ref[0]
    ay1 = anchors_ref[1]
    ax2 = anchors_ref[2]
    ay2 = anchors_ref[3]
    dx = deltas_ref[0, 0]
    dy = deltas_ref[0, 1]
    dw = deltas_ref[0, 2]
    dh = deltas_ref[0, 3]

    widths = ax2 - ax1 + 1.0
    heights = ay2 - ay1 + 1.0
    ctr_x = ax1 + 0.5 * widths
    ctr_y = ay1 + 0.5 * heights
    pcx = dx * widths + ctr_x
    pcy = dy * heights + ctr_y
    pw = jnp.exp(dw) * widths
    ph = jnp.exp(dh) * heights
    x1 = jnp.clip(pcx - 0.5 * pw, 0.0, w_im - 1.0)
    y1 = jnp.clip(pcy - 0.5 * ph, 0.0, h_im - 1.0)
    x2 = jnp.clip(pcx + 0.5 * pw, 0.0, w_im - 1.0)
    y2 = jnp.clip(pcy + 0.5 * ph, 0.0, h_im - 1.0)

    table_ref[0, 0] = x1
    table_ref[0, 1] = y1
    table_ref[0, 2] = x2
    table_ref[0, 3] = y2

    row_i = jax.lax.broadcasted_iota(jnp.int32, (_ROWS, 128), 0)
    col_i = jax.lax.broadcasted_iota(jnp.int32, (_ROWS, 128), 1)
    nphys = row_i * 128 + col_i
    a_id = nphys // 2500
    p_id = nphys - a_id * 2500
    nref = jnp.where(nphys < _N_REAL, p_id * 9 + a_id, nphys)

    arrays = [s, nref]

    def partner(x, j):
        if j < 128:
            sel = (col_i & j) == 0
            return jnp.where(sel, pltpu.roll(x, 128 - j, 1),
                             pltpu.roll(x, j, 1))
        dj = j // 128
        sel = (row_i & dj) == 0
        return jnp.where(sel, pltpu.roll(x, _ROWS - dj, 0),
                         pltpu.roll(x, dj, 0))

    k = 2
    while k <= _N_SORT:
        j = k // 2
        while j >= 1:
            ps = [partner(a, j) for a in arrays]
            if j < 128:
                is_lower = (col_i & j) == 0
            else:
                is_lower = (row_i & (j // 128)) == 0
            if k < 128:
                dir_up = (col_i & k) == 0
            elif k < _N_SORT:
                dir_up = (row_i & (k // 128)) == 0
            else:
                dir_up = jnp.full((_ROWS, 128), True)
            lt = (arrays[0] > ps[0]) | ((arrays[0] == ps[0]) &
                                        (arrays[1] < ps[1]))
            take_small = is_lower == dir_up
            keep_self = lt == take_small
            arrays = [jnp.where(keep_self, a, p) for a, p in zip(arrays, ps)]
            j //= 2
        k *= 2

    sn = arrays[1][:_SROWS]
    sp = sn // 9
    sa = sn - sp * 9
    sidx = jnp.minimum(sa * 2500 + sp, _N_REAL - 1)
    base = (i * 4) * _N_TAB
    for c in range(4):
        idx_ref[0, c] = base + c * _N_TAB + sidx


def _make_sc_gather(b):
    n_idx = b * 4 * _S
    nw = 32
    per_w = n_idx // nw
    chunks = per_w // 128
    mesh = plsc.VectorSubcoreMesh(core_axis_name="c", subcore_axis_name="s")

    @functools.partial(
        pl.kernel,
        out_type=jax.ShapeDtypeStruct((n_idx,), jnp.float32),
        mesh=mesh,
        scratch_types=[
            pltpu.VMEM((per_w,), jnp.int32),
            pltpu.VMEM((per_w,), jnp.float32),
            pltpu.SemaphoreType.DMA,
        ],
    )
    def sc_gather(table_hbm, idx_hbm, out_hbm, idx_v, rows_v, sem):
        wid = lax.axis_index("s") * 2 + lax.axis_index("c")
        base = wid * per_w
        pltpu.sync_copy(idx_hbm.at[pl.ds(base, per_w)], idx_v)
        copies = [
            pltpu.async_copy(table_hbm.at[idx_v.at[pl.ds(j * 128, 128)]],
                             rows_v.at[pl.ds(j * 128, 128)], sem)
            for j in range(chunks)
        ]
        for cp in copies:
            cp.wait()
        pltpu.sync_copy(rows_v, out_hbm.at[pl.ds(base, per_w)])

    return sc_gather


def _nms_kernel(boxes_ref, out_ref, keep_ref, cnt_ref):
    x1 = boxes_ref[0, 0]
    y1 = boxes_ref[0, 1]
    x2 = boxes_ref[0, 2]
    y2 = boxes_ref[0, 3]
    areas = (x2 - x1 + 1.0) * (y2 - y1 + 1.0)

    keep_ref[...] = jnp.zeros((_SROWS, 128), jnp.float32)
    cnt_ref[0] = 0.0

    sub_i = jax.lax.broadcasted_iota(jnp.int32, (128, 128), 0)
    lan_i = jax.lax.broadcasted_iota(jnp.int32, (128, 128), 1)
    eye = (sub_i == lan_i).astype(jnp.float32)
    strict_lower = (lan_i < sub_i).astype(jnp.float32)

    x1b = x1[None]
    y1b = y1[None]
    x2b = x2[None]
    y2b = y2[None]
    areab = areas[None]

    def to_col(row):
        return jax.lax.dot_general(eye, row, (((1,), (1,)), ((), ())),
                                   preferred_element_type=jnp.float32)

    def to_row(col):
        return jax.lax.dot_general(col, eye, (((0,), (0,)), ((), ())),
                                   preferred_element_type=jnp.float32)

    def matvec(m, v):
        return jax.lax.dot_general(m, v, (((1,), (0,)), ((), ())),
                                   preferred_element_type=jnp.float32)

    def tile_body(t, carry):
        @pl.when(cnt_ref[0] < float(_POST_NMS))
        def _process():
            rx1 = boxes_ref[0, 0, pl.ds(t, 1), :]
            ry1 = boxes_ref[0, 1, pl.ds(t, 1), :]
            rx2 = boxes_ref[0, 2, pl.ds(t, 1), :]
            ry2 = boxes_ref[0, 3, pl.ds(t, 1), :]
            rar = (rx2 - rx1 + 1.0) * (ry2 - ry1 + 1.0)
            cx1 = to_col(rx1)
            cy1 = to_col(ry1)
            cx2 = to_col(rx2)
            cy2 = to_col(ry2)
            car = to_col(rar)

            xx1 = jnp.maximum(cx1[:, :, None], x1b)
            yy1 = jnp.maximum(cy1[:, :, None], y1b)
            xx2 = jnp.minimum(cx2[:, :, None], x2b)
            yy2 = jnp.minimum(cy2[:, :, None], y2b)
            w = jnp.maximum(0.0, xx2 - xx1 + 1.0)
            h = jnp.maximum(0.0, yy2 - yy1 + 1.0)
            inter = w * h
            iou = inter / (car[:, :, None] + areab - inter)
            m3 = (iou > _NMS_THRESH).astype(jnp.float32)

            supp = jnp.max(m3 * keep_ref[...][None], axis=(1, 2),
                           keepdims=True)[:, :, 0]
            g = t * 128 + jax.lax.broadcasted_iota(jnp.int32, (128, 1), 0)
            alive = ((supp == 0.0) & (g < _PRE_NMS)).astype(jnp.float32)

            ixx1 = jnp.maximum(cx1, rx1)
            iyy1 = jnp.maximum(cy1, ry1)
            ixx2 = jnp.minimum(cx2, rx2)
            iyy2 = jnp.minimum(cy2, ry2)
            iw = jnp.maximum(0.0, ixx2 - ixx1 + 1.0)
            ih = jnp.maximum(0.0, iyy2 - iyy1 + 1.0)
            iin = iw * ih
            iself = iin / (car + rar - iin)
            ov = jnp.where(iself > _NMS_THRESH, strict_lower, 0.0)

            lo0 = alive * jnp.where(matvec(ov, alive) > 0.0, 0.0, 1.0)
            up0 = alive

            def fx_cond(lu):
                lo, up = lu
                return jnp.sum(up - lo) > 0.0

            def fx_body(lu):
                lo, _ = lu
                up = alive * jnp.where(matvec(ov, lo) > 0.0, 0.0, 1.0)
                lo2 = alive * jnp.where(matvec(ov, up) > 0.0, 0.0, 1.0)
                return lo2, up

            lo, _ = jax.lax.while_loop(fx_cond, fx_body, (lo0, up0))

            keep_ref[pl.ds(t, 1), :] = to_row(lo)
            cnt_ref[0] = cnt_ref[0] + jnp.sum(lo)

        return carry

    jax.lax.fori_loop(0, _SROWS, tile_body, 0, unroll=False)

    keep = keep_ref[...]
    up128 = (sub_i < lan_i).astype(jnp.float32)
    inrow = jax.lax.dot_general(keep, up128, (((1,), (0,)), ((), ())),
                                preferred_element_type=jnp.float32)
    rsub = jax.lax.broadcasted_iota(jnp.int32, (_SROWS, _SROWS), 0)
    rlan = jax.lax.broadcasted_iota(jnp.int32, (_SROWS, _SROWS), 1)
    low48 = (rlan < rsub).astype(jnp.float32)
    ones_col = jnp.ones((128, 1), jnp.float32)
    rowsum = jax.lax.dot_general(keep, ones_col, (((1,), (0,)), ((), ())),
                                 preferred_element_type=jnp.float32)
    rowoff = jax.lax.dot_general(low48, rowsum, (((1,), (0,)), ((), ())),
                                 preferred_element_type=jnp.float32)
    rank = inrow + rowoff

    p_col = jax.lax.broadcasted_iota(jnp.int32, (_OUT_ROWS, 1, 1), 0)
    onehot = jnp.where((rank[None] == p_col.astype(jnp.float32)) &
                       (keep[None] > 0.0), 1.0, 0.0)

    def reduce_coord(c3):
        return jnp.sum(onehot * c3, axis=(1, 2), keepdims=True)[:, :, 0]

    ox1 = reduce_coord(x1b)
    oy1 = reduce_coord(y1b)
    ox2 = reduce_coord(x2b)
    oy2 = reduce_coord(y2b)
    bcol = jnp.full((_OUT_ROWS, 1),
                    pl.program_id(0).astype(jnp.float32))

    lane = jax.lax.broadcasted_iota(jnp.int32, (_OUT_ROWS, 128), 1)
    out = jnp.zeros((_OUT_ROWS, 128), jnp.float32)
    for c, col in enumerate([bcol, ox1, oy1, ox2, oy2]):
        out = jnp.where(lane == c, jnp.broadcast_to(col, (_OUT_ROWS, 128)),
                        out)
    out_ref[0] = out


def kernel(rpn_cls_prob, rpn_bbox_pred, im_info):
    b, _, fh, fw = rpn_cls_prob.shape
    n_real = fh * fw * 9

    scores = rpn_cls_prob[:, 9:, :, :].reshape(b, n_real)
    scores = jnp.pad(scores, ((0, 0), (0, _N_SORT - n_real)),
                     constant_values=-1.0)
    scores = scores.reshape(b, _ROWS, 128)

    deltas = rpn_bbox_pred.reshape(b, 9, 4, fh * fw)
    deltas = jnp.transpose(deltas, (0, 2, 1, 3)).reshape(b, 4, n_real)
    deltas = jnp.pad(deltas, ((0, 0), (0, 0), (0, _N_TAB - n_real)))
    deltas = deltas.reshape(b, 4, _TROWS, 128)

    anchors = _full_anchors(fh, fw)
    anchors = anchors.reshape(fh * fw, 9, 4).transpose(1, 0, 2)
    anchors = anchors.reshape(n_real, 4)
    anchors = np.pad(anchors, ((0, _N_TAB - n_real), (0, 0)))
    anchors = jnp.asarray(anchors.T.reshape(4, _TROWS, 128))

    imhw = im_info[:, :2]

    table, flatidx = pl.pallas_call(
        _sort_kernel,
        grid=(b,),
        in_specs=[
            pl.BlockSpec((1, _ROWS, 128), lambda i: (i, 0, 0)),
            pl.BlockSpec((1, 4, _TROWS, 128), lambda i: (i, 0, 0, 0)),
            pl.BlockSpec((4, _TROWS, 128), lambda i: (0, 0, 0)),
            pl.BlockSpec((4, 2), lambda i: (0, 0),
                         memory_space=pltpu.SMEM),
        ],
        out_specs=(
            pl.BlockSpec((1, 4, _TROWS, 128), lambda i: (i, 0, 0, 0)),
            pl.BlockSpec((1, 4, _SROWS, 128), lambda i: (i, 0, 0, 0)),
        ),
        out_shape=(
            jax.ShapeDtypeStruct((b, 4, _TROWS, 128), jnp.float32),
            jax.ShapeDtypeStruct((b, 4, _SROWS, 128), jnp.int32),
        ),
    )(scores, deltas, anchors, imhw)

    gathered = _make_sc_gather(b)(table.reshape(-1), flatidx.reshape(-1))
    boxes = gathered.reshape(b, 4, _SROWS, 128)

    out = pl.pallas_call(
        _nms_kernel,
        grid=(b,),
        in_specs=[
            pl.BlockSpec((1, 4, _SROWS, 128), lambda i: (i, 0, 0, 0)),
        ],
        out_specs=pl.BlockSpec((1, _OUT_ROWS, 128), lambda i: (i, 0, 0)),
        out_shape=jax.ShapeDtypeStruct((b, _OUT_ROWS, 128), jnp.float32),
        scratch_shapes=[
            pltpu.VMEM((_SROWS, 128), jnp.float32),
            pltpu.SMEM((1,), jnp.float32),
        ],
    )(boxes)

    return out[:, :_POST_NMS, :5]
